# baseline (speedup 1.0000x reference)
import jax
import jax.numpy as jnp
from jax import lax
from jax.experimental import pallas as pl
from jax.experimental.pallas import tpu as pltpu
from jax.experimental.pallas import tpu_sc as plsc

E = 1600000
D = 32
V = 1024
C = 400
F = 3 * C
STREAMS = tuple((o, min(128, F - o)) for o in range(0, F, 128))
UNROLL = 8

_info = plsc.get_sparse_core_info()
NC, NS = _info.num_cores, _info.num_subcores
NW = NC * NS
PER_W = E // NW
NCHUNK = PER_W // C
NPAIR = (NCHUNK - 1) // 2


def _body(flat_hbm, w_hbm, out_hbm, iv, rv, ov, isem, gsem, osem):
    w = lax.axis_index("s") * NC + lax.axis_index("c")
    w_base = w * PER_W

    lane = lax.iota(jnp.int32, 16)
    offv = tuple(((lane + 16 * t) % 3) * V for t in range(3))

    def chunk_base(k):
        return w_base + jnp.minimum(k, NCHUNK - 1) * C

    def load_idx(b, k):
        pltpu.async_copy(flat_hbm.at[pl.ds(chunk_base(k) * 3, F)], iv.at[b],
                         isem.at[b])

    def wait_idx(b):
        pltpu.make_async_copy(flat_hbm.at[pl.ds(0, F)], iv.at[b],
                              isem.at[b]).wait()

    def add_offsets(b):
        @plsc.parallel_loop(0, F, 48, unroll=4)
        def _(i):
            for t in range(3):
                sl = pl.ds(i + 16 * t, 16)
                iv[b, sl] = iv[b, sl] + offv[t]

    def fire_gathers(b):
        for off, n in STREAMS:
            pltpu.async_copy(w_hbm.at[iv.at[b].at[pl.ds(off, n)]],
                             rv.at[b].at[pl.ds(off, n)], gsem.at[b])

    def wait_gathers(b):
        for off, n in STREAMS:
            pltpu.make_async_copy(w_hbm.at[iv.at[b].at[pl.ds(off, n)]],
                                  rv.at[b].at[pl.ds(off, n)], gsem.at[b]).wait()

    def compute_store(b, k, first):
        @pl.when(jnp.logical_not(first))
        def _():
            pltpu.make_async_copy(ov.at[b], out_hbm.at[pl.ds(0, C * D)],
                                  osem.at[b]).wait()

        @plsc.parallel_loop(0, C, 1, unroll=UNROLL)
        def _(i):
            for h in range(D // 16):
                sl = pl.ds(i * D + h * 16, 16)
                hs = pl.ds(h * 16, 16)
                ov[b, sl] = (rv[b, 3 * i, hs] + rv[b, 3 * i + 1, hs]
                             + rv[b, 3 * i + 2, hs])

        pltpu.async_copy(ov.at[b], out_hbm.at[pl.ds(chunk_base(k) * D, C * D)],
                         osem.at[b])

    def stage_gathers(b):
        wait_idx(b)
        add_offsets(b)
        fire_gathers(b)

    load_idx(0, 0)
    stage_gathers(0)
    load_idx(1, 1)

    def pair_body(p, carry):
        k = p * 2
        stage_gathers(1)
        wait_gathers(0)
        compute_store(0, k, p == 0)
        load_idx(0, k + 2)
        stage_gathers(0)
        wait_gathers(1)
        compute_store(1, k + 1, p == 0)
        load_idx(1, k + 3)
        return carry

    lax.fori_loop(0, NPAIR, pair_body, 0)

    wait_gathers(0)
    compute_store(0, NCHUNK - 1, False)
    wait_idx(1)
    for b in range(2):
        pltpu.make_async_copy(ov.at[b], out_hbm.at[pl.ds(0, C * D)],
                              osem.at[b]).wait()


def kernel(edge_attr, W0, W1, W2):
    flat = edge_attr.reshape(-1)
    w_all = jnp.concatenate([W0, W1, W2], axis=0)
    run = pl.kernel(
        _body,
        out_type=jax.ShapeDtypeStruct((E * D,), jnp.float32),
        mesh=plsc.VectorSubcoreMesh(core_axis_name="c", subcore_axis_name="s"),
        compiler_params=pltpu.CompilerParams(use_tc_tiling_on_sc=False),
        scratch_types=[
            pltpu.VMEM((2, F), jnp.int32),
            pltpu.VMEM((2, F, D), jnp.float32),
            pltpu.VMEM((2, C * D), jnp.float32),
            pltpu.SemaphoreType.DMA((2,)),
            pltpu.SemaphoreType.DMA((2,)),
            pltpu.SemaphoreType.DMA((2,)),
        ],
    )
    return run(flat, w_all).reshape(E, D)


# speedup vs baseline: 6.8861x; 6.8861x over previous
import jax
import jax.numpy as jnp
from jax import lax
from jax.experimental import pallas as pl
from jax.experimental.pallas import tpu as pltpu
from jax.experimental.pallas import tpu_sc as plsc

E = 1600000
D = 32
V = 1024
TBL = 3 * V * D
C = 80
UNROLL = 1

_info = plsc.get_sparse_core_info()
NC, NS = _info.num_cores, _info.num_subcores
NW = NC * NS
PER_W = E // NW
NCHUNK = PER_W // C


def _body(i0_hbm, i1_hbm, i2_hbm, w_hbm, out_hbm,
          wv, ova, ovb, ia0, ia1, ia2, ib0, ib1, ib2, isem, osem):
    w = lax.axis_index("s") * NC + lax.axis_index("c")
    w_base = w * PER_W
    idx_srcs = (i0_hbm, i1_hbm, i2_hbm)
    ivs = ((ia0, ia1, ia2), (ib0, ib1, ib2))
    ovs = (ova, ovb)

    pltpu.sync_copy(w_hbm, wv)

    def chunk_base(k):
        return w_base + jnp.minimum(k, NCHUNK - 1) * C

    def load_idx(b, k):
        base = chunk_base(k)
        for t in range(3):
            pltpu.async_copy(idx_srcs[t].at[pl.ds(base, C)], ivs[b][t],
                             isem.at[b])

    def wait_idx(b):
        for t in range(3):
            pltpu.make_async_copy(idx_srcs[t].at[pl.ds(w_base, C)],
                                  ivs[b][t], isem.at[b]).wait()

    def compute_store(b, k, first):
        ov = ovs[b]
        @pl.when(jnp.logical_not(first))
        def _():
            pltpu.make_async_copy(ov, out_hbm.at[pl.ds(w_base, C)],
                                  osem.at[b]).wait()

        @plsc.parallel_loop(0, C, 16, unroll=UNROLL)
        def _(g):
            v0 = ivs[b][0][pl.ds(g, 16)] * D
            v1 = ivs[b][1][pl.ds(g, 16)] * D + V * D
            v2 = ivs[b][2][pl.ds(g, 16)] * D + 2 * V * D
            for lane in range(16):
                a0, a1, a2 = v0[lane], v1[lane], v2[lane]
                for h in range(D // 16):
                    ov[g + lane, pl.ds(h * 16, 16)] = (
                        wv[pl.ds(a0 + h * 16, 16)]
                        + wv[pl.ds(a1 + h * 16, 16)]
                        + wv[pl.ds(a2 + h * 16, 16)])

        pltpu.async_copy(ov, out_hbm.at[pl.ds(chunk_base(k), C)], osem.at[b])

    load_idx(0, 0)
    load_idx(1, 1)

    def pair_body(p, carry):
        k = p * 2
        wait_idx(0)
        compute_store(0, k, p == 0)
        load_idx(0, k + 2)
        wait_idx(1)
        compute_store(1, k + 1, p == 0)
        load_idx(1, k + 3)
        return carry

    lax.fori_loop(0, NCHUNK // 2, pair_body, 0)

    wait_idx(0)
    compute_store(0, NCHUNK - 1, False)
    wait_idx(1)
    for b in range(2):
        pltpu.make_async_copy(ovs[b], out_hbm.at[pl.ds(w_base, C)],
                              osem.at[b]).wait()


def kernel(edge_attr, W0, W1, W2):
    idx0 = edge_attr[:, 0]
    idx1 = edge_attr[:, 1]
    idx2 = edge_attr[:, 2]
    w_flat = jnp.concatenate([W0, W1, W2], axis=0).reshape(-1)
    run = pl.kernel(
        _body,
        out_type=jax.ShapeDtypeStruct((E, D), jnp.float32),
        mesh=plsc.VectorSubcoreMesh(core_axis_name="c", subcore_axis_name="s"),
        compiler_params=pltpu.CompilerParams(use_tc_tiling_on_sc=True),
        scratch_types=[
            pltpu.VMEM((TBL,), jnp.float32),
            pltpu.VMEM((C, D), jnp.float32),
            pltpu.VMEM((C, D), jnp.float32),
            pltpu.VMEM((C,), jnp.int32),
            pltpu.VMEM((C,), jnp.int32),
            pltpu.VMEM((C,), jnp.int32),
            pltpu.VMEM((C,), jnp.int32),
            pltpu.VMEM((C,), jnp.int32),
            pltpu.VMEM((C,), jnp.int32),
            pltpu.SemaphoreType.DMA((2,)),
            pltpu.SemaphoreType.DMA((2,)),
        ],
    )
    return run(idx0, idx1, idx2, w_flat)
